# baseline (speedup 1.0000x reference)
import functools

import jax
import jax.numpy as jnp
from jax.experimental import pallas as pl
from jax.experimental.pallas import tpu as pltpu

_BM1 = 256
_BR = 256
_P1 = 40
_NPAD = _P1 * _BM1


def _fused_kernel(ti_ref, adj_ref, x_ref, W1_ref, b1_ref, W2_ref, Wf_ref,
                  b2_ref, bf_ref, adj_hbm, out_ref,
                  z1_s, w2f_s, z2b_s, gath_s, sem):
    i = pl.program_id(0)
    nsteps = pl.num_programs(0)

    @pl.when(i == 0)
    def _():
        z1_s[...] = jnp.dot(x_ref[...], W1_ref[...],
                            preferred_element_type=jnp.float32
                            ).astype(jnp.bfloat16)
        w2f_s[...] = jnp.dot(W2_ref[...], Wf_ref[...],
                             preferred_element_type=jnp.float32)

    @pl.when(i < _P1)
    def _():
        ab = adj_ref[...].astype(jnp.bfloat16)
        t = jnp.dot(ab, z1_s[...], preferred_element_type=jnp.float32)
        h = jnp.maximum(t + b1_ref[...], 0.0)
        z2 = jnp.dot(h, w2f_s[...], preferred_element_type=jnp.float32)
        z2b_s[pl.ds(i * _BM1, _BM1), :] = z2.astype(jnp.bfloat16)

    def issue(buf, base):
        def body(r, carry):
            i0 = base + 8 * r
            pltpu.make_async_copy(adj_hbm.at[ti_ref[i0 + 0]],
                                  gath_s.at[buf, 8 * r + 0], sem.at[buf, 0]).start()
            pltpu.make_async_copy(adj_hbm.at[ti_ref[i0 + 1]],
                                  gath_s.at[buf, 8 * r + 1], sem.at[buf, 1]).start()
            pltpu.make_async_copy(adj_hbm.at[ti_ref[i0 + 2]],
                                  gath_s.at[buf, 8 * r + 2], sem.at[buf, 2]).start()
            pltpu.make_async_copy(adj_hbm.at[ti_ref[i0 + 3]],
                                  gath_s.at[buf, 8 * r + 3], sem.at[buf, 3]).start()
            pltpu.make_async_copy(adj_hbm.at[ti_ref[i0 + 4]],
                                  gath_s.at[buf, 8 * r + 4], sem.at[buf, 4]).start()
            pltpu.make_async_copy(adj_hbm.at[ti_ref[i0 + 5]],
                                  gath_s.at[buf, 8 * r + 5], sem.at[buf, 5]).start()
            pltpu.make_async_copy(adj_hbm.at[ti_ref[i0 + 6]],
                                  gath_s.at[buf, 8 * r + 6], sem.at[buf, 6]).start()
            pltpu.make_async_copy(adj_hbm.at[ti_ref[i0 + 7]],
                                  gath_s.at[buf, 8 * r + 7], sem.at[buf, 7]).start()
            return carry
        jax.lax.fori_loop(0, _BR // 8, body, 0, unroll=4)

    @pl.when(i == _P1 - 4)
    def _():
        issue(0, 0)

    @pl.when(i == _P1 - 2)
    def _():
        issue(1, _BR)

    def drain(buf):
        def body(r, carry):
            pltpu.make_async_copy(adj_hbm.at[0], gath_s.at[buf, 0],
                                  sem.at[buf, 0]).wait()
            pltpu.make_async_copy(adj_hbm.at[0], gath_s.at[buf, 1],
                                  sem.at[buf, 1]).wait()
            pltpu.make_async_copy(adj_hbm.at[0], gath_s.at[buf, 2],
                                  sem.at[buf, 2]).wait()
            pltpu.make_async_copy(adj_hbm.at[0], gath_s.at[buf, 3],
                                  sem.at[buf, 3]).wait()
            pltpu.make_async_copy(adj_hbm.at[0], gath_s.at[buf, 4],
                                  sem.at[buf, 4]).wait()
            pltpu.make_async_copy(adj_hbm.at[0], gath_s.at[buf, 5],
                                  sem.at[buf, 5]).wait()
            pltpu.make_async_copy(adj_hbm.at[0], gath_s.at[buf, 6],
                                  sem.at[buf, 6]).wait()
            pltpu.make_async_copy(adj_hbm.at[0], gath_s.at[buf, 7],
                                  sem.at[buf, 7]).wait()
            return carry
        jax.lax.fori_loop(0, _BR // 8, body, 0, unroll=4)

    n = adj_hbm.shape[0]

    def finish(buf, j):
        drain(buf)
        acc = jnp.dot(gath_s[buf].astype(jnp.bfloat16), z2b_s[0:n, :],
                      preferred_element_type=jnp.float32)
        bias = jnp.dot(b2_ref[...], Wf_ref[...],
                       preferred_element_type=jnp.float32) + bf_ref[...]
        o = acc + bias
        m = jnp.max(o, axis=1, keepdims=True)
        lse = m + jnp.log(jnp.sum(jnp.exp(o - m), axis=1, keepdims=True))
        out_ref[...] = o - lse
        @pl.when(j + 2 < nsteps - _P1)
        def _():
            issue(buf, (j + 2) * _BR)

    @pl.when((i >= _P1) & ((i - _P1) % 2 == 0))
    def _():
        finish(0, i - _P1)

    @pl.when((i >= _P1) & ((i - _P1) % 2 == 1))
    def _():
        finish(1, i - _P1)


def kernel(x, adj, type_index, non_zero_index, non_zero_value,
           W1, b1, W2, b2, Wf, bf):
    n, nfeat = x.shape
    nhid2 = W1.shape[1]
    nhid = W2.shape[1]
    ncls = Wf.shape[1]
    t = type_index.shape[0]

    b1r = b1.reshape(1, nhid2)
    b2r = b2.reshape(1, nhid)
    bfr = bf.reshape(1, ncls)

    p2 = t // _BR
    grid_spec = pltpu.PrefetchScalarGridSpec(
        num_scalar_prefetch=1,
        grid=(_P1 + p2,),
        in_specs=[
            pl.BlockSpec((_BM1, n), lambda i, ti: (jnp.minimum(i, _P1 - 1), 0)),
            pl.BlockSpec((n, nfeat), lambda i, ti: (0, 0)),
            pl.BlockSpec((nfeat, nhid2), lambda i, ti: (0, 0)),
            pl.BlockSpec((1, nhid2), lambda i, ti: (0, 0)),
            pl.BlockSpec((nhid2, nhid), lambda i, ti: (0, 0)),
            pl.BlockSpec((nhid, ncls), lambda i, ti: (0, 0)),
            pl.BlockSpec((1, nhid), lambda i, ti: (0, 0)),
            pl.BlockSpec((1, ncls), lambda i, ti: (0, 0)),
            pl.BlockSpec(memory_space=pl.ANY),
        ],
        out_specs=pl.BlockSpec((_BR, ncls),
                               lambda i, ti: (jnp.maximum(i - _P1, 0), 0)),
        scratch_shapes=[pltpu.VMEM((n, nhid2), jnp.bfloat16),
                        pltpu.VMEM((nhid2, ncls), jnp.float32),
                        pltpu.VMEM((_NPAD, ncls), jnp.bfloat16),
                        pltpu.VMEM((2, _BR, n), jnp.float32),
                        pltpu.SemaphoreType.DMA((2, 8))],
    )
    out = pl.pallas_call(
        _fused_kernel,
        grid_spec=grid_spec,
        out_shape=jax.ShapeDtypeStruct((t, ncls), jnp.float32),
    )(type_index, adj, x, W1, b1r, W2, Wf, b2r, bfr, adj)
    return out


# speedup vs baseline: 1.0143x; 1.0143x over previous
import functools

import jax
import jax.numpy as jnp
from jax.experimental import pallas as pl
from jax.experimental.pallas import tpu as pltpu

_BM1 = 256
_BR = 256
_P1 = 40
_NPAD = _P1 * _BM1


def _fused_kernel(ti_ref, adj_ref, x_ref, W1_ref, b1_ref, W2_ref, Wf_ref,
                  b2_ref, bf_ref, adj_hbm, out_ref,
                  z1_s, w2f_s, z2b_s, gath_s, sem):
    i = pl.program_id(0)
    nsteps = pl.num_programs(0)

    @pl.when(i == 0)
    def _():
        z1_s[...] = jnp.dot(x_ref[...], W1_ref[...],
                            preferred_element_type=jnp.float32
                            ).astype(jnp.bfloat16)
        w2f_s[...] = jnp.dot(W2_ref[...], Wf_ref[...],
                             preferred_element_type=jnp.float32)

    @pl.when(i < _P1)
    def _():
        ab = adj_ref[...].astype(jnp.bfloat16)
        t = jnp.dot(ab, z1_s[...], preferred_element_type=jnp.float32)
        h = jnp.maximum(t + b1_ref[...], 0.0)
        z2 = jnp.dot(h, w2f_s[...], preferred_element_type=jnp.float32)
        z2b_s[pl.ds(i * _BM1, _BM1), :] = z2.astype(jnp.bfloat16)

    def issue(buf, base):
        def body(r, carry):
            i0 = base + 8 * r
            pltpu.make_async_copy(adj_hbm.at[ti_ref[i0 + 0]],
                                  gath_s.at[buf, 8 * r + 0], sem.at[buf, 0]).start()
            pltpu.make_async_copy(adj_hbm.at[ti_ref[i0 + 1]],
                                  gath_s.at[buf, 8 * r + 1], sem.at[buf, 1]).start()
            pltpu.make_async_copy(adj_hbm.at[ti_ref[i0 + 2]],
                                  gath_s.at[buf, 8 * r + 2], sem.at[buf, 2]).start()
            pltpu.make_async_copy(adj_hbm.at[ti_ref[i0 + 3]],
                                  gath_s.at[buf, 8 * r + 3], sem.at[buf, 3]).start()
            pltpu.make_async_copy(adj_hbm.at[ti_ref[i0 + 4]],
                                  gath_s.at[buf, 8 * r + 4], sem.at[buf, 4]).start()
            pltpu.make_async_copy(adj_hbm.at[ti_ref[i0 + 5]],
                                  gath_s.at[buf, 8 * r + 5], sem.at[buf, 5]).start()
            pltpu.make_async_copy(adj_hbm.at[ti_ref[i0 + 6]],
                                  gath_s.at[buf, 8 * r + 6], sem.at[buf, 6]).start()
            pltpu.make_async_copy(adj_hbm.at[ti_ref[i0 + 7]],
                                  gath_s.at[buf, 8 * r + 7], sem.at[buf, 7]).start()
            return carry
        jax.lax.fori_loop(0, _BR // 8, body, 0, unroll=4)

    @pl.when(i == _P1 - 2)
    def _():
        issue(0, 0)

    @pl.when(i == _P1 - 1)
    def _():
        issue(1, _BR)

    def drain(buf):
        def body(r, carry):
            pltpu.make_async_copy(adj_hbm.at[0], gath_s.at[buf, 0],
                                  sem.at[buf, 0]).wait()
            pltpu.make_async_copy(adj_hbm.at[0], gath_s.at[buf, 1],
                                  sem.at[buf, 1]).wait()
            pltpu.make_async_copy(adj_hbm.at[0], gath_s.at[buf, 2],
                                  sem.at[buf, 2]).wait()
            pltpu.make_async_copy(adj_hbm.at[0], gath_s.at[buf, 3],
                                  sem.at[buf, 3]).wait()
            pltpu.make_async_copy(adj_hbm.at[0], gath_s.at[buf, 4],
                                  sem.at[buf, 4]).wait()
            pltpu.make_async_copy(adj_hbm.at[0], gath_s.at[buf, 5],
                                  sem.at[buf, 5]).wait()
            pltpu.make_async_copy(adj_hbm.at[0], gath_s.at[buf, 6],
                                  sem.at[buf, 6]).wait()
            pltpu.make_async_copy(adj_hbm.at[0], gath_s.at[buf, 7],
                                  sem.at[buf, 7]).wait()
            return carry
        jax.lax.fori_loop(0, _BR // 8, body, 0, unroll=4)

    n = adj_hbm.shape[0]

    def finish(buf, j):
        drain(buf)
        acc = jnp.dot(gath_s[buf].astype(jnp.bfloat16), z2b_s[0:n, :],
                      preferred_element_type=jnp.float32)
        bias = jnp.dot(b2_ref[...], Wf_ref[...],
                       preferred_element_type=jnp.float32) + bf_ref[...]
        o = acc + bias
        m = jnp.max(o, axis=1, keepdims=True)
        lse = m + jnp.log(jnp.sum(jnp.exp(o - m), axis=1, keepdims=True))
        out_ref[...] = o - lse
        @pl.when(j + 2 < nsteps - _P1)
        def _():
            issue(buf, (j + 2) * _BR)

    @pl.when((i >= _P1) & ((i - _P1) % 2 == 0))
    def _():
        finish(0, i - _P1)

    @pl.when((i >= _P1) & ((i - _P1) % 2 == 1))
    def _():
        finish(1, i - _P1)


def kernel(x, adj, type_index, non_zero_index, non_zero_value,
           W1, b1, W2, b2, Wf, bf):
    n, nfeat = x.shape
    nhid2 = W1.shape[1]
    nhid = W2.shape[1]
    ncls = Wf.shape[1]
    t = type_index.shape[0]

    b1r = b1.reshape(1, nhid2)
    b2r = b2.reshape(1, nhid)
    bfr = bf.reshape(1, ncls)

    p2 = t // _BR
    grid_spec = pltpu.PrefetchScalarGridSpec(
        num_scalar_prefetch=1,
        grid=(_P1 + p2,),
        in_specs=[
            pl.BlockSpec((_BM1, n), lambda i, ti: (jnp.minimum(i, _P1 - 1), 0)),
            pl.BlockSpec((n, nfeat), lambda i, ti: (0, 0)),
            pl.BlockSpec((nfeat, nhid2), lambda i, ti: (0, 0)),
            pl.BlockSpec((1, nhid2), lambda i, ti: (0, 0)),
            pl.BlockSpec((nhid2, nhid), lambda i, ti: (0, 0)),
            pl.BlockSpec((nhid, ncls), lambda i, ti: (0, 0)),
            pl.BlockSpec((1, nhid), lambda i, ti: (0, 0)),
            pl.BlockSpec((1, ncls), lambda i, ti: (0, 0)),
            pl.BlockSpec(memory_space=pl.ANY),
        ],
        out_specs=pl.BlockSpec((_BR, ncls),
                               lambda i, ti: (jnp.maximum(i - _P1, 0), 0)),
        scratch_shapes=[pltpu.VMEM((n, nhid2), jnp.bfloat16),
                        pltpu.VMEM((nhid2, ncls), jnp.float32),
                        pltpu.VMEM((_NPAD, ncls), jnp.bfloat16),
                        pltpu.VMEM((2, _BR, n), jnp.float32),
                        pltpu.SemaphoreType.DMA((2, 8))],
    )
    out = pl.pallas_call(
        _fused_kernel,
        grid_spec=grid_spec,
        out_shape=jax.ShapeDtypeStruct((t, ncls), jnp.float32),
    )(type_index, adj, x, W1, b1r, W2, Wf, b2r, bfr, adj)
    return out
